# Initial kernel scaffold; baseline (speedup 1.0000x reference)
import jax
import jax.numpy as jnp
from jax.experimental import pallas as pl


def kernel(x, emb_weight):
    raise NotImplementedError("write your pallas kernel here")


# speedup vs baseline: 4.9045x; 4.9045x over previous
import functools

import jax
import jax.numpy as jnp
from jax import lax
from jax.experimental import pallas as pl
from jax.experimental.pallas import tpu as pltpu
from jax.experimental.pallas import tpu_sc as plsc

D_MODEL = 32
N_TOKENS = 16384 * 200
NUM_WORKERS = 32
B_PER_W = N_TOKENS // NUM_WORKERS
CHUNK = 1600
N_CHUNKS = B_PER_W // CHUNK

_mesh = plsc.VectorSubcoreMesh(core_axis_name="c", subcore_axis_name="s")


@functools.partial(
    pl.kernel,
    mesh=_mesh,
    out_type=jax.ShapeDtypeStruct((N_TOKENS, D_MODEL), jnp.float32),
    scratch_types=[
        pltpu.VMEM((CHUNK,), jnp.int32),
        pltpu.VMEM((CHUNK, D_MODEL), jnp.float32),
        pltpu.SemaphoreType.DMA,
    ],
    compiler_params=pltpu.CompilerParams(use_tc_tiling_on_sc=False),
)
def _emb_lookup(idx_hbm, table_hbm, out_hbm, idx_v, rows_v, sem):
    wid = lax.axis_index("s") * 2 + lax.axis_index("c")
    wbase = wid * B_PER_W

    def body(j, carry):
        base = wbase + j * CHUNK
        pltpu.sync_copy(idx_hbm.at[pl.ds(base, CHUNK)], idx_v)
        pltpu.async_copy(table_hbm.at[idx_v], rows_v, sem).wait()
        pltpu.sync_copy(rows_v, out_hbm.at[pl.ds(base, CHUNK)])
        return carry

    lax.fori_loop(0, N_CHUNKS, body, 0)


def kernel(x, emb_weight):
    flat = x.reshape(-1)
    out = _emb_lookup(flat, emb_weight)
    return out.reshape(x.shape + (emb_weight.shape[1],))


# speedup vs baseline: 5.0524x; 1.0301x over previous
import functools

import jax
import jax.numpy as jnp
from jax import lax
from jax.experimental import pallas as pl
from jax.experimental.pallas import tpu as pltpu
from jax.experimental.pallas import tpu_sc as plsc

D_MODEL = 32
N_TOKENS = 16384 * 200
NUM_WORKERS = 32
B_PER_W = N_TOKENS // NUM_WORKERS
CHUNK = 1600
N_CHUNKS = B_PER_W // CHUNK

_mesh = plsc.VectorSubcoreMesh(core_axis_name="c", subcore_axis_name="s")


@functools.partial(
    pl.kernel,
    mesh=_mesh,
    out_type=jax.ShapeDtypeStruct((N_TOKENS, D_MODEL), jnp.float32),
    scratch_types=[
        pltpu.VMEM((CHUNK,), jnp.int32),
        pltpu.VMEM((CHUNK,), jnp.int32),
        pltpu.VMEM((CHUNK, D_MODEL), jnp.float32),
        pltpu.VMEM((CHUNK, D_MODEL), jnp.float32),
        pltpu.SemaphoreType.DMA,
        pltpu.SemaphoreType.DMA,
        pltpu.SemaphoreType.DMA,
        pltpu.SemaphoreType.DMA,
        pltpu.SemaphoreType.DMA,
        pltpu.SemaphoreType.DMA,
    ],
    compiler_params=pltpu.CompilerParams(use_tc_tiling_on_sc=False),
)
def _emb_lookup(idx_hbm, table_hbm, out_hbm,
                i0, i1, r0, r1, si0, si1, sg0, sg1, so0, so1):
    wid = lax.axis_index("s") * 2 + lax.axis_index("c")
    wbase = wid * B_PER_W
    iv = (i0, i1)
    rv = (r0, r1)
    si = (si0, si1)
    sg = (sg0, sg1)
    so = (so0, so1)

    def start_idx(jc, b):
        pltpu.async_copy(idx_hbm.at[pl.ds(wbase + jc * CHUNK, CHUNK)],
                         iv[b], si[b])

    def wait_idx(b):
        pltpu.make_async_copy(idx_hbm.at[pl.ds(0, CHUNK)], iv[b], si[b]).wait()

    def start_gather(b):
        pltpu.async_copy(table_hbm.at[iv[b]], rv[b], sg[b])

    def wait_gather(b):
        pltpu.make_async_copy(out_hbm.at[pl.ds(0, CHUNK)], rv[b], sg[b]).wait()

    def start_out(jc, b):
        pltpu.async_copy(rv[b], out_hbm.at[pl.ds(wbase + jc * CHUNK, CHUNK)],
                         so[b])

    def wait_out(b):
        pltpu.make_async_copy(rv[b], out_hbm.at[pl.ds(0, CHUNK)], so[b]).wait()

    start_idx(0, 0)
    wait_idx(0)
    start_gather(0)
    start_idx(1, 1)
    wait_idx(1)
    start_gather(1)
    wait_gather(0)
    start_out(0, 0)
    start_idx(2, 0)

    @pl.loop(2, N_CHUNKS, step=2)
    def _(j):
        for b in (0, 1):
            jc = j + b
            wait_idx(b)
            wait_out(b)
            start_gather(b)
            wait_gather(1 - b)
            start_out(jc - 1, 1 - b)

            @pl.when(jc + 1 < N_CHUNKS)
            def _():
                start_idx(jc + 1, 1 - b)

    wait_gather(1)
    start_out(N_CHUNKS - 1, 1)
    wait_out(0)
    wait_out(1)


def kernel(x, emb_weight):
    flat = x.reshape(-1)
    out = _emb_lookup(flat, emb_weight)
    return out.reshape(x.shape + (emb_weight.shape[1],))
